# baseline (speedup 1.0000x reference)
import math

import jax
import jax.numpy as jnp
from jax import lax
from jax.experimental import pallas as pl
from jax.experimental.pallas import tpu as pltpu
from jax.experimental.pallas import tpu_sc as plsc

def _cp(tc_tiling):
    cp = pltpu.CompilerParams(use_tc_tiling_on_sc=tc_tiling)
    if "needs_layout_passes" in pltpu.CompilerParams.__dataclass_fields__:
        import dataclasses
        cp = dataclasses.replace(cp, needs_layout_passes=False)
    return cp


NC = 2
NS = 16
NW = NC * NS
V = 1000000
D = 64
SCALE = math.sqrt(D)

K0_CHUNKS_MAIN = 7808
K0_PER_W = K0_CHUNKS_MAIN // NW


def _table_format(t_cm, tail):
    mesh = plsc.VectorSubcoreMesh(core_axis_name="c", subcore_axis_name="s")

    @pl.kernel(
        out_type=jax.ShapeDtypeStruct((V * D,), jnp.float32),
        mesh=mesh,
        compiler_params=_cp(True),
        scratch_types=[
            pltpu.VMEM((2, 64, 128), jnp.float32),
            pltpu.VMEM((8192,), jnp.float32),
            pltpu.VMEM((8192,), jnp.float32),
            pltpu.VMEM((64, D), jnp.float32),
            pltpu.SemaphoreType.DMA((2,)),
            pltpu.SemaphoreType.DMA((2,)),
        ],
    )
    def k0(tcm_hbm, tail_hbm, out_hbm, vbuf, sbuf0, sbuf1, tbuf, gsem,
           ssem):
        sbufs = (sbuf0, sbuf1)
        wid = lax.axis_index("c") * NS + lax.axis_index("s")
        j0 = wid * K0_PER_W
        iota = lax.iota(jnp.int32, 16)

        def in_copies(j, b):
            return [
                pltpu.make_async_copy(
                    tcm_hbm.at[pl.ds(cg * 8, 8), pl.ds(j * 128, 128)],
                    vbuf.at[b, pl.ds(cg * 8, 8), :], gsem.at[b])
                for cg in range(8)
            ]

        def store_copy(j, b):
            return pltpu.make_async_copy(
                sbufs[b], out_hbm.at[pl.ds(j * 8192, 8192)], ssem.at[b])

        rbase = [(iota + r0) * 64 for r0 in range(0, 128, 16)]

        def transpose_chunk(b):
            v = vbuf.at[b]
            s = sbufs[b]

            @plsc.parallel_loop(0, 64, unroll=8)
            def _(c):
                for k in range(8):
                    vec = v.at[c, pl.ds(k * 16, 16)][...]
                    plsc.store_scatter(s, [rbase[k] + c], vec)

        def process(j, b, wait_store, issue_next):
            for cp in in_copies(j, b):
                cp.wait()
            if wait_store:
                store_copy(j, b).wait()
            transpose_chunk(b)
            if issue_next:
                for cp in in_copies(j + 2, b):
                    cp.start()
            store_copy(j, b).start()

        for b in range(2):
            for cp in in_copies(j0 + b, b):
                cp.start()
        for b in range(2):
            process(j0 + b, b, wait_store=False, issue_next=True)

        @pl.loop(1, K0_PER_W // 2 - 1)
        def _(g):
            jj = j0 + 2 * g
            for b in range(2):
                process(jj + b, b, wait_store=True, issue_next=True)

        jj = j0 + K0_PER_W - 2
        for b in range(2):
            process(jj + b, b, wait_store=True, issue_next=False)
        for b in range(2):
            store_copy(jj + b, b).wait()

        @pl.when(wid < 4)
        def _():
            j = K0_CHUNKS_MAIN + wid
            for cp in in_copies(j, 0):
                cp.start()
            for cp in in_copies(j, 0):
                cp.wait()
            transpose_chunk(0)
            store_copy(j, 0).start()
            store_copy(j, 0).wait()

        @pl.when(wid == 4)
        def _():
            pltpu.sync_copy(tail_hbm, tbuf)
            for i in range(64):
                for k in range(D // 16):
                    sbuf0.at[pl.ds(i * D + k * 16, 16)][...] = (
                        tbuf.at[i, pl.ds(k * 16, 16)][...])
            pltpu.sync_copy(sbuf0.at[pl.ds(0, 64 * D)],
                            out_hbm.at[pl.ds((V - 64) * D, 64 * D)])

    return k0(t_cm, tail)


SB = 25
BB = 32


def _gather_scale(tok6, tab_rows):
    mesh = plsc.VectorSubcoreMesh(core_axis_name="c", subcore_axis_name="s")

    @pl.kernel(
        out_type=jax.ShapeDtypeStruct((200 * 8 * BB * 8 * 128,), jnp.float32),
        mesh=mesh,
        compiler_params=_cp(False),
        scratch_types=[
            pltpu.VMEM((SB, 8, 128), jnp.int32),
            pltpu.VMEM((2, 128, D), jnp.float32),
            pltpu.VMEM((2, 8192), jnp.float32),
            pltpu.SemaphoreType.DMA((2,)),
            pltpu.SemaphoreType.DMA((2,)),
        ],
    )
    def k2(tok_hbm, tab_hbm, out_hbm, idx_v, rows, stage, gsem, ssem):
        wid = lax.axis_index("c") * NS + lax.axis_index("s")
        iota = lax.iota(jnp.int32, 16)

        pltpu.sync_copy(tok_hbm.at[:, wid], idx_v)

        def gather(s, b):
            return pltpu.make_async_copy(
                tab_hbm.at[idx_v.at[lax.shift_right_logical(s, 3),
                                    lax.bitwise_and(s, 7)]],
                rows.at[b], gsem.at[b])

        def stores(s, b):
            return [
                pltpu.make_async_copy(
                    stage.at[b, pl.ds(cg * 1024, 1024)],
                    out_hbm.at[pl.ds((s * 8 + cg) * BB * 1024 + wid * 1024,
                                     1024)],
                    ssem.at[b])
                for cg in range(8)
            ]

        cbase = [(iota + c0) * 128 for c0 in range(0, D, 16)]

        def transpose_scale(s, b):
            r_ref = rows.at[b]
            s_ref = stage.at[b]

            @plsc.parallel_loop(0, 128, unroll=8)
            def _(t):
                for k in range(D // 16):
                    vec = r_ref.at[t, pl.ds(k * 16, 16)][...] * SCALE
                    plsc.store_scatter(s_ref, [cbase[k] + t], vec)

        def process(s, b, wait_store, issue_next):
            if issue_next:
                gather(s + 1, 1 - b).start()
            gather(s, b).wait()
            if wait_store:
                for cp in stores(s, b):
                    cp.wait()
            transpose_scale(s, b)
            for cp in stores(s, b):
                cp.start()

        gather(0, 0).start()
        process(0, 0, wait_store=False, issue_next=True)
        process(1, 1, wait_store=False, issue_next=True)

        @pl.loop(1, 99)
        def _(g):
            s0 = 2 * g
            process(s0, 0, wait_store=True, issue_next=True)
            process(s0 + 1, 1, wait_store=True, issue_next=True)

        process(198, 0, wait_store=True, issue_next=True)
        process(199, 1, wait_store=True, issue_next=False)
        for b, s in ((0, 198), (1, 199)):
            for cp in stores(s, b):
                cp.wait()

    return k2(tok6, tab_rows)


def kernel(token_ids, embedding_table):
    bsz, seq = token_ids.shape
    tail = embedding_table[V - 64:, :]
    tab_flat = _table_format(embedding_table.T, tail)
    tab_rows = tab_flat.reshape(V, D)
    tok6 = (token_ids.astype(jnp.int32).T
            .reshape(SB, 8, BB, 128).transpose(0, 2, 1, 3))
    out5 = _gather_scale(tok6, tab_rows).reshape(200, 8, BB, 8, 128)
    return out5.transpose(2, 4, 0, 1, 3).reshape(bsz, seq, D)


# speedup vs baseline: 1.2839x; 1.2839x over previous
import math

import jax
import jax.numpy as jnp
from jax import lax
from jax.experimental import pallas as pl
from jax.experimental.pallas import tpu as pltpu
from jax.experimental.pallas import tpu_sc as plsc

NC = 2
NS = 16
NW = NC * NS
LANES = 16
NBUF = 4
LEAD = 2
SPLIT = 104


def _sc_embedding_lookup(tok, table, scale):
    bsz, seq = tok.shape
    d = table.shape[1]
    rows_per_w = bsz // NW
    mesh = plsc.VectorSubcoreMesh(core_axis_name="c", subcore_axis_name="s")

    @pl.kernel(
        out_type=jax.ShapeDtypeStruct((bsz, seq, d), jnp.float32),
        mesh=mesh,
        compiler_params=pltpu.CompilerParams(use_tc_tiling_on_sc=False),
        scratch_types=[
            pltpu.VMEM((rows_per_w, seq), jnp.int32),
            pltpu.VMEM((NBUF, seq, d), jnp.float32),
            pltpu.SemaphoreType.DMA((NBUF,)),
            pltpu.SemaphoreType.DMA((NBUF,)),
        ],
    )
    def k(tok_hbm, table_hbm, out_hbm, idx_v, gbuf, gsem, ssem):
        wid = lax.axis_index("c") * NS + lax.axis_index("s")
        row0 = wid * rows_per_w

        pltpu.sync_copy(tok_hbm.at[pl.ds(row0, rows_per_w)], idx_v)

        def gather_copies(i, b):
            return (
                pltpu.make_async_copy(
                    table_hbm.at[idx_v.at[i, pl.ds(0, SPLIT)]],
                    gbuf.at[b, pl.ds(0, SPLIT)], gsem.at[b]),
                pltpu.make_async_copy(
                    table_hbm.at[idx_v.at[i, pl.ds(SPLIT, seq - SPLIT)]],
                    gbuf.at[b, pl.ds(SPLIT, seq - SPLIT)], gsem.at[b]),
            )

        def store_copy(i, b):
            return pltpu.make_async_copy(
                gbuf.at[b], out_hbm.at[row0 + i], ssem.at[b])

        def issue_gather(i, b):
            for cp in gather_copies(i, b):
                cp.start()

        def wait_gather(i, b):
            for cp in gather_copies(i, b):
                cp.wait()

        def scale_rows(b):
            g = gbuf.at[b]

            @pl.loop(0, seq, step=8)
            def _(r):
                for dr in range(8):
                    for c in range(d // LANES):
                        sl = (pl.ds(r + dr, 1), pl.ds(c * LANES, LANES))
                        g.at[sl][...] = g.at[sl][...] * scale

        def process(i, k_slot, refill):
            wait_gather(i, k_slot)
            scale_rows(k_slot)
            store_copy(i, k_slot).start()
            if refill:
                nxt = (k_slot + LEAD) % NBUF
                store_copy(i - LEAD, nxt).wait()
                issue_gather(i + LEAD, nxt)

        for b in range(NBUF):
            issue_gather(b, b)
        for kk in range(NBUF):
            process(kk, kk, refill=(kk >= LEAD))

        @pl.loop(1, rows_per_w // NBUF - 1)
        def _(grp):
            i0 = grp * NBUF
            for kk in range(NBUF):
                process(i0 + kk, kk, refill=True)

        i0 = rows_per_w - NBUF
        for kk in range(NBUF):
            process(i0 + kk, kk, refill=(kk < LEAD))

        for kk in range(NBUF):
            store_copy(i0 + kk, kk).wait()

    return k(tok, table)


def kernel(token_ids, embedding_table):
    bsz, seq = token_ids.shape
    d = embedding_table.shape[1]
    assert bsz % NW == 0 and (bsz // NW) % NBUF == 0 and d % LANES == 0
    scale = math.sqrt(d)
    return _sc_embedding_lookup(
        token_ids.astype(jnp.int32), embedding_table, scale)
